# baseline (speedup 1.0000x reference)
import jax
import jax.numpy as jnp
from jax.experimental import pallas as pl
from jax.experimental.pallas import tpu as pltpu

_D_IN = 2048
_BASE = 256
_NUM_LATENTS = 65536
_TB = 128
_IB = 128


def _body(x_ref, w_ref, b_ref, out_ref, h_ref):
    i = pl.program_id(1)

    @pl.when(i == 0)
    def _compute_h():
        h = jax.lax.dot_general(
            x_ref[...], w_ref[...],
            dimension_numbers=(((1,), (1,)), ((), ())),
            preferred_element_type=jnp.float32,
        )
        h_ref[...] = h + b_ref[...]

    x1 = h_ref[:, pl.ds(i * _IB, _IB)]
    x2 = h_ref[:, _BASE:]
    out_ref[...] = x1[:, :, None] + x2[:, None, :]


def kernel(x, W, b):
    n_tok = x.shape[0]
    grid = (n_tok // _TB, _BASE // _IB)
    out3 = pl.pallas_call(
        _body,
        grid=grid,
        in_specs=[
            pl.BlockSpec((_TB, _D_IN), lambda t, i: (t, 0)),
            pl.BlockSpec((2 * _BASE, _D_IN), lambda t, i: (0, 0)),
            pl.BlockSpec((1, 2 * _BASE), lambda t, i: (0, 0)),
        ],
        out_specs=pl.BlockSpec((_TB, _IB, _BASE), lambda t, i: (t, i, 0)),
        out_shape=jax.ShapeDtypeStruct((n_tok, _BASE, _BASE), jnp.float32),
        scratch_shapes=[pltpu.VMEM((_TB, 2 * _BASE), jnp.float32)],
    )(x, W, b.reshape(1, 2 * _BASE))
    return out3.reshape(n_tok, _BASE * _BASE)[:, :_NUM_LATENTS]


# speedup vs baseline: 3.3024x; 3.3024x over previous
import jax
import jax.numpy as jnp
from jax.experimental import pallas as pl

_D_IN = 2048
_BASE = 256
_NUM_LATENTS = 65536
_TB = 64


def _body(x_ref, w_ref, b_ref, out_ref):
    h = jax.lax.dot_general(
        x_ref[...], w_ref[...],
        dimension_numbers=(((1,), (1,)), ((), ())),
        preferred_element_type=jnp.float32,
    ) + b_ref[...]
    x1 = h[:, :_BASE]
    x2 = h[:, _BASE:]
    for k in range(_BASE):
        out_ref[:, k * _BASE:(k + 1) * _BASE] = x1[:, k:k + 1] + x2


def kernel(x, W, b):
    n_tok = x.shape[0]
    out = pl.pallas_call(
        _body,
        grid=(n_tok // _TB,),
        in_specs=[
            pl.BlockSpec((_TB, _D_IN), lambda t: (t, 0)),
            pl.BlockSpec((2 * _BASE, _D_IN), lambda t: (0, 0)),
            pl.BlockSpec((1, 2 * _BASE), lambda t: (0, 0)),
        ],
        out_specs=pl.BlockSpec((_TB, _BASE * _BASE), lambda t: (t, 0)),
        out_shape=jax.ShapeDtypeStruct((n_tok, _BASE * _BASE), jnp.float32),
    )(x, W, b.reshape(1, 2 * _BASE))
    return out[:, :_NUM_LATENTS]
